# baseline (speedup 1.0000x reference)
import functools

import jax
import jax.numpy as jnp
from jax.experimental import pallas as pl
from jax.experimental.pallas import tpu as pltpu

N_EMB = 1024
EMB_DIM = 64
TOKEN_BLOCK = 1024


def _vq_kernel(x_ref, w_ref, w2_ref, q_ref, loss_ref, perp_ref, use_ref,
               counts_ref, dacc_ref, *, n_tokens, grid):
    i = pl.program_id(0)

    @pl.when(i == 0)
    def _init():
        counts_ref[...] = jnp.zeros_like(counts_ref)
        dacc_ref[...] = jnp.zeros_like(dacc_ref)

    x_blk = x_ref[...]
    w = w_ref[...]
    K = w.shape[0]

    x2 = jnp.sum(x_blk ** 2, axis=1, keepdims=True)
    mm = jax.lax.dot_general(x_blk, w, (((1,), (1,)), ((), ())),
                             preferred_element_type=jnp.float32)
    d = (w2_ref[...] + x2) - 2.0 * mm

    dmin = jnp.min(d, axis=1, keepdims=True)
    iota = jax.lax.broadcasted_iota(jnp.int32, d.shape, 1)
    idxc = jnp.min(jnp.where(d == dmin, iota, K), axis=1, keepdims=True)

    onehot = (iota == idxc).astype(jnp.float32)
    q = jax.lax.dot_general(onehot, w, (((1,), (0,)), ((), ())),
                            preferred_element_type=jnp.float32)
    q_ref[...] = x_blk + (q - x_blk)

    counts_ref[...] += jnp.sum(onehot, axis=0, keepdims=True)
    dacc_ref[...] += jnp.sum(onehot * d, axis=0, keepdims=True)

    @pl.when(i == grid - 1)
    def _finalize():
        counts = counts_ref[...]
        mse = jnp.sum(dacc_ref[...], axis=1, keepdims=True) \
            * (1.0 / float(n_tokens * EMB_DIM))
        loss_ref[...] = mse + 2.0 * mse
        avg = counts * (1.0 / float(n_tokens))
        ent = jnp.sum(avg * jnp.log(avg + 1e-10), axis=1, keepdims=True)
        perp_ref[...] = jnp.exp(-ent)
        use_ref[...] = jnp.sum((counts >= 1.0).astype(jnp.float32),
                               axis=1, keepdims=True)


@jax.jit
def kernel(x, weight):
    K, D = weight.shape
    x_flat = x.reshape(-1, D)
    n_tokens = x_flat.shape[0]
    grid = n_tokens // TOKEN_BLOCK
    w2_row = jnp.sum(weight ** 2, axis=1)[None, :]

    kfn = functools.partial(_vq_kernel, n_tokens=n_tokens, grid=grid)
    out_shapes = (
        jax.ShapeDtypeStruct((n_tokens, D), jnp.float32),
        jax.ShapeDtypeStruct((1, 1), jnp.float32),
        jax.ShapeDtypeStruct((1, 1), jnp.float32),
        jax.ShapeDtypeStruct((1, 1), jnp.float32),
    )
    scalar_spec = pl.BlockSpec((1, 1), lambda i: (0, 0))
    q_st, loss, perp, use = pl.pallas_call(
        kfn,
        grid=(grid,),
        in_specs=[
            pl.BlockSpec((TOKEN_BLOCK, D), lambda i: (i, 0)),
            pl.BlockSpec((K, D), lambda i: (0, 0)),
            pl.BlockSpec((1, K), lambda i: (0, 0)),
        ],
        out_specs=(
            pl.BlockSpec((TOKEN_BLOCK, D), lambda i: (i, 0)),
            scalar_spec, scalar_spec, scalar_spec,
        ),
        out_shape=out_shapes,
        scratch_shapes=[
            pltpu.VMEM((1, K), jnp.float32),
            pltpu.VMEM((1, K), jnp.float32),
        ],
    )(x_flat, weight, w2_row)

    return (q_st.reshape(x.shape), loss[0, 0], perp[0, 0], use[0, 0])


# speedup vs baseline: 1.0931x; 1.0931x over previous
import functools

import jax
import jax.numpy as jnp
from jax.experimental import pallas as pl
from jax.experimental.pallas import tpu as pltpu

N_EMB = 1024
EMB_DIM = 64
TOKEN_BLOCK = 1024


def _vq_kernel(x_ref, w_ref, w2_ref, q_ref, loss_ref, perp_ref, use_ref,
               counts_ref, dacc_ref, *, n_tokens, grid):
    i = pl.program_id(0)

    @pl.when(i == 0)
    def _init():
        counts_ref[...] = jnp.zeros_like(counts_ref)
        dacc_ref[...] = jnp.zeros_like(dacc_ref)

    x_blk = x_ref[...]
    w = w_ref[...]
    K = w.shape[0]
    TB = x_blk.shape[0]

    x2 = jnp.sum(x_blk ** 2, axis=1, keepdims=True)
    mm2 = jax.lax.dot_general(x_blk * (-2.0), w, (((1,), (1,)), ((), ())),
                              preferred_element_type=jnp.float32)
    d = (w2_ref[...] + x2) + mm2

    dmin = jnp.min(d, axis=1, keepdims=True)
    iota = jax.lax.broadcasted_iota(jnp.int32, d.shape, 1)
    idxc = jnp.min(jnp.where(d == dmin, iota, K), axis=1, keepdims=True)

    onehot = (iota == idxc).astype(jnp.float32)
    q = jax.lax.dot_general(onehot, w, (((1,), (0,)), ((), ())),
                            preferred_element_type=jnp.float32)
    q_ref[...] = x_blk + (q - x_blk)

    ones_row = jnp.ones((1, TB), dtype=jnp.float32)
    counts_ref[...] += jax.lax.dot_general(
        ones_row, onehot, (((1,), (0,)), ((), ())),
        preferred_element_type=jnp.float32)
    dacc_ref[...] += dmin

    @pl.when(i == grid - 1)
    def _finalize():
        counts = counts_ref[...]
        mse = jnp.sum(dacc_ref[...], axis=0, keepdims=True) \
            * (1.0 / float(n_tokens * EMB_DIM))
        loss_ref[...] = mse + 2.0 * mse
        avg = counts * (1.0 / float(n_tokens))
        ent = jnp.sum(avg * jnp.log(avg + 1e-10), axis=1, keepdims=True)
        perp_ref[...] = jnp.exp(-ent)
        use_ref[...] = jnp.sum((counts >= 1.0).astype(jnp.float32),
                               axis=1, keepdims=True)


@jax.jit
def kernel(x, weight):
    K, D = weight.shape
    x_flat = x.reshape(-1, D)
    n_tokens = x_flat.shape[0]
    grid = n_tokens // TOKEN_BLOCK
    w2_row = jnp.sum(weight ** 2, axis=1)[None, :]

    kfn = functools.partial(_vq_kernel, n_tokens=n_tokens, grid=grid)
    out_shapes = (
        jax.ShapeDtypeStruct((n_tokens, D), jnp.float32),
        jax.ShapeDtypeStruct((1, 1), jnp.float32),
        jax.ShapeDtypeStruct((1, 1), jnp.float32),
        jax.ShapeDtypeStruct((1, 1), jnp.float32),
    )
    scalar_spec = pl.BlockSpec((1, 1), lambda i: (0, 0))
    q_st, loss, perp, use = pl.pallas_call(
        kfn,
        grid=(grid,),
        in_specs=[
            pl.BlockSpec((TOKEN_BLOCK, D), lambda i: (i, 0)),
            pl.BlockSpec((K, D), lambda i: (0, 0)),
            pl.BlockSpec((1, K), lambda i: (0, 0)),
        ],
        out_specs=(
            pl.BlockSpec((TOKEN_BLOCK, D), lambda i: (i, 0)),
            scalar_spec, scalar_spec, scalar_spec,
        ),
        out_shape=out_shapes,
        scratch_shapes=[
            pltpu.VMEM((1, K), jnp.float32),
            pltpu.VMEM((TOKEN_BLOCK, 1), jnp.float32),
        ],
    )(x_flat, weight, w2_row)

    return (q_st.reshape(x.shape), loss[0, 0], perp[0, 0], use[0, 0])


# speedup vs baseline: 1.1453x; 1.0477x over previous
import functools

import jax
import jax.numpy as jnp
from jax.experimental import pallas as pl
from jax.experimental.pallas import tpu as pltpu

N_EMB = 1024
EMB_DIM = 64
TOKEN_BLOCK = 2048


def _vq_kernel(x_ref, w_ref, w2_ref, q_ref, loss_ref, perp_ref, use_ref,
               counts_ref, dacc_ref, *, n_tokens, grid):
    i = pl.program_id(0)

    @pl.when(i == 0)
    def _init():
        counts_ref[...] = jnp.zeros_like(counts_ref)
        dacc_ref[...] = jnp.zeros_like(dacc_ref)

    x_blk = x_ref[...]
    w = w_ref[...]
    K = w.shape[0]
    TB = x_blk.shape[0]

    x2 = jnp.sum(x_blk ** 2, axis=1, keepdims=True)
    mm2 = jax.lax.dot_general(x_blk * (-2.0), w, (((1,), (1,)), ((), ())),
                              preferred_element_type=jnp.float32)
    d = (w2_ref[...] + x2) + mm2

    dmin = jnp.min(d, axis=1, keepdims=True)
    iota = jax.lax.broadcasted_iota(jnp.int32, d.shape, 1)
    idxc = jnp.min(jnp.where(d == dmin, iota, K), axis=1, keepdims=True)

    onehot = (iota == idxc).astype(jnp.float32)
    q = jax.lax.dot_general(onehot, w, (((1,), (0,)), ((), ())),
                            preferred_element_type=jnp.float32)
    q_ref[...] = x_blk + (q - x_blk)

    ones_row = jnp.ones((1, TB), dtype=jnp.float32)
    counts_ref[...] += jax.lax.dot_general(
        ones_row, onehot, (((1,), (0,)), ((), ())),
        preferred_element_type=jnp.float32)
    dacc_ref[...] += dmin

    @pl.when(i == grid - 1)
    def _finalize():
        counts = counts_ref[...]
        mse = jnp.sum(dacc_ref[...], axis=0, keepdims=True) \
            * (1.0 / float(n_tokens * EMB_DIM))
        loss_ref[...] = mse + 2.0 * mse
        avg = counts * (1.0 / float(n_tokens))
        ent = jnp.sum(avg * jnp.log(avg + 1e-10), axis=1, keepdims=True)
        perp_ref[...] = jnp.exp(-ent)
        use_ref[...] = jnp.sum((counts >= 1.0).astype(jnp.float32),
                               axis=1, keepdims=True)


@jax.jit
def kernel(x, weight):
    K, D = weight.shape
    x_flat = x.reshape(-1, D)
    n_tokens = x_flat.shape[0]
    grid = n_tokens // TOKEN_BLOCK
    w2_row = jnp.sum(weight ** 2, axis=1)[None, :]

    kfn = functools.partial(_vq_kernel, n_tokens=n_tokens, grid=grid)
    out_shapes = (
        jax.ShapeDtypeStruct((n_tokens, D), jnp.float32),
        jax.ShapeDtypeStruct((1, 1), jnp.float32),
        jax.ShapeDtypeStruct((1, 1), jnp.float32),
        jax.ShapeDtypeStruct((1, 1), jnp.float32),
    )
    scalar_spec = pl.BlockSpec((1, 1), lambda i: (0, 0))
    q_st, loss, perp, use = pl.pallas_call(
        kfn,
        grid=(grid,),
        in_specs=[
            pl.BlockSpec((TOKEN_BLOCK, D), lambda i: (i, 0)),
            pl.BlockSpec((K, D), lambda i: (0, 0)),
            pl.BlockSpec((1, K), lambda i: (0, 0)),
        ],
        out_specs=(
            pl.BlockSpec((TOKEN_BLOCK, D), lambda i: (i, 0)),
            scalar_spec, scalar_spec, scalar_spec,
        ),
        out_shape=out_shapes,
        scratch_shapes=[
            pltpu.VMEM((1, K), jnp.float32),
            pltpu.VMEM((TOKEN_BLOCK, 1), jnp.float32),
        ],
    )(x_flat, weight, w2_row)

    return (q_st.reshape(x.shape), loss[0, 0], perp[0, 0], use[0, 0])


# speedup vs baseline: 1.1502x; 1.0043x over previous
import functools

import jax
import jax.numpy as jnp
from jax.experimental import pallas as pl
from jax.experimental.pallas import tpu as pltpu

N_EMB = 1024
EMB_DIM = 64
TOKEN_BLOCK = 2048


def _vq_kernel(x_ref, w_ref, w2_ref, q_ref, loss_ref, perp_ref, use_ref,
               counts_ref, dacc_ref, *, n_tokens, grid):
    i = pl.program_id(0)

    @pl.when(i == 0)
    def _init():
        counts_ref[...] = jnp.zeros_like(counts_ref)
        dacc_ref[...] = jnp.zeros_like(dacc_ref)

    x3 = x_ref[...]
    x_blk = x3.reshape(x3.shape[0] * x3.shape[1], x3.shape[2])
    w = w_ref[...]
    K = w.shape[0]
    TB = x_blk.shape[0]

    x2 = jnp.sum(x_blk ** 2, axis=1, keepdims=True)
    mm2 = jax.lax.dot_general(x_blk * (-2.0), w, (((1,), (1,)), ((), ())),
                              preferred_element_type=jnp.float32)
    d = (w2_ref[...] + x2) + mm2

    dmin = jnp.min(d, axis=1, keepdims=True)
    iota = jax.lax.broadcasted_iota(jnp.int32, d.shape, 1)
    idxc = jnp.min(jnp.where(d == dmin, iota, K), axis=1, keepdims=True)

    onehot = (iota == idxc).astype(jnp.float32)
    q = jax.lax.dot_general(onehot, w, (((1,), (0,)), ((), ())),
                            preferred_element_type=jnp.float32)
    q_ref[...] = (x_blk + (q - x_blk)).reshape(x3.shape)

    ones_row = jnp.ones((1, TB), dtype=jnp.float32)
    counts_ref[...] += jax.lax.dot_general(
        ones_row, onehot, (((1,), (0,)), ((), ())),
        preferred_element_type=jnp.float32)
    dacc_ref[...] += dmin

    @pl.when(i == grid - 1)
    def _finalize():
        counts = counts_ref[...]
        mse = jnp.sum(dacc_ref[...], axis=0, keepdims=True) \
            * (1.0 / float(n_tokens * EMB_DIM))
        loss_ref[...] = mse + 2.0 * mse
        avg = counts * (1.0 / float(n_tokens))
        ent = jnp.sum(avg * jnp.log(avg + 1e-10), axis=1, keepdims=True)
        perp_ref[...] = jnp.exp(-ent)
        use_ref[...] = jnp.sum((counts >= 1.0).astype(jnp.float32),
                               axis=1, keepdims=True)


@jax.jit
def kernel(x, weight):
    K, D = weight.shape
    B, S, _ = x.shape
    n_tokens = B * S
    rows = TOKEN_BLOCK // S
    grid = n_tokens // TOKEN_BLOCK
    w2_row = jnp.sum(weight ** 2, axis=1)[None, :]

    kfn = functools.partial(_vq_kernel, n_tokens=n_tokens, grid=grid)
    out_shapes = (
        jax.ShapeDtypeStruct((B, S, D), jnp.float32),
        jax.ShapeDtypeStruct((1, 1), jnp.float32),
        jax.ShapeDtypeStruct((1, 1), jnp.float32),
        jax.ShapeDtypeStruct((1, 1), jnp.float32),
    )
    scalar_spec = pl.BlockSpec((1, 1), lambda i: (0, 0))
    q_st, loss, perp, use = pl.pallas_call(
        kfn,
        grid=(grid,),
        in_specs=[
            pl.BlockSpec((rows, S, D), lambda i: (i, 0, 0)),
            pl.BlockSpec((K, D), lambda i: (0, 0)),
            pl.BlockSpec((1, K), lambda i: (0, 0)),
        ],
        out_specs=(
            pl.BlockSpec((rows, S, D), lambda i: (i, 0, 0)),
            scalar_spec, scalar_spec, scalar_spec,
        ),
        out_shape=out_shapes,
        scratch_shapes=[
            pltpu.VMEM((1, K), jnp.float32),
            pltpu.VMEM((TOKEN_BLOCK, 1), jnp.float32),
        ],
    )(x, weight, w2_row)

    return (q_st, loss[0, 0], perp[0, 0], use[0, 0])


# speedup vs baseline: 1.6828x; 1.4630x over previous
import functools

import jax
import jax.numpy as jnp
from jax.experimental import pallas as pl
from jax.experimental.pallas import tpu as pltpu

N_EMB = 1024
EMB_DIM = 64
ROWS = 2


def _vq_kernel(x_ref, w_ref, q_ref, loss_ref, perp_ref, use_ref,
               counts_ref, dacc_ref, *, n_tokens, grid):
    i = pl.program_id(0)

    @pl.when(i == 0)
    def _init():
        counts_ref[...] = jnp.zeros_like(counts_ref)
        dacc_ref[...] = jnp.zeros_like(dacc_ref)

    w = w_ref[...]
    K = w.shape[0]
    w2 = jnp.sum(w ** 2, axis=1, keepdims=True)
    wneg2 = w * (-2.0)
    ones_col = jnp.ones((x_ref.shape[2], 1), dtype=jnp.float32)

    for r in range(x_ref.shape[0]):
        xb = x_ref[r]
        x2 = jnp.sum(xb ** 2, axis=0, keepdims=True)
        mm2 = jax.lax.dot_general(wneg2, xb, (((1,), (0,)), ((), ())),
                                  preferred_element_type=jnp.float32)
        d = (w2 + x2) + mm2

        dmin = jnp.min(d, axis=0, keepdims=True)
        iota = jax.lax.broadcasted_iota(jnp.int32, d.shape, 0)
        idxr = jnp.min(jnp.where(d == dmin, iota, K), axis=0, keepdims=True)

        onehot = (iota == idxr).astype(jnp.float32)
        q = jax.lax.dot_general(w, onehot, (((0,), (0,)), ((), ())),
                                preferred_element_type=jnp.float32)
        q_ref[r] = xb + (q - xb)

        counts_ref[...] += jax.lax.dot_general(
            onehot, ones_col, (((1,), (0,)), ((), ())),
            preferred_element_type=jnp.float32)
        dacc_ref[...] += dmin

    @pl.when(i == grid - 1)
    def _finalize():
        counts = counts_ref[...]
        mse = jnp.sum(dacc_ref[...], axis=1, keepdims=True) \
            * (1.0 / float(n_tokens * EMB_DIM))
        loss_ref[...] = mse + 2.0 * mse
        avg = counts * (1.0 / float(n_tokens))
        ent = jnp.sum(avg * jnp.log(avg + 1e-10), axis=0, keepdims=True)
        perp_ref[...] = jnp.exp(-ent)
        use_ref[...] = jnp.sum((counts >= 1.0).astype(jnp.float32),
                               axis=0, keepdims=True)


@jax.jit
def kernel(x, weight):
    K, D = weight.shape
    B, S, _ = x.shape
    n_tokens = B * S
    grid = B // ROWS
    xt = jnp.swapaxes(x, 1, 2)

    kfn = functools.partial(_vq_kernel, n_tokens=n_tokens, grid=grid)
    out_shapes = (
        jax.ShapeDtypeStruct((B, D, S), jnp.float32),
        jax.ShapeDtypeStruct((1, 1), jnp.float32),
        jax.ShapeDtypeStruct((1, 1), jnp.float32),
        jax.ShapeDtypeStruct((1, 1), jnp.float32),
    )
    scalar_spec = pl.BlockSpec((1, 1), lambda i: (0, 0))
    q_t, loss, perp, use = pl.pallas_call(
        kfn,
        grid=(grid,),
        in_specs=[
            pl.BlockSpec((ROWS, D, S), lambda i: (i, 0, 0)),
            pl.BlockSpec((K, D), lambda i: (0, 0)),
        ],
        out_specs=(
            pl.BlockSpec((ROWS, D, S), lambda i: (i, 0, 0)),
            scalar_spec, scalar_spec, scalar_spec,
        ),
        out_shape=out_shapes,
        scratch_shapes=[
            pltpu.VMEM((K, 1), jnp.float32),
            pltpu.VMEM((1, S), jnp.float32),
        ],
    )(xt, weight)

    return (jnp.swapaxes(q_t, 1, 2), loss[0, 0], perp[0, 0], use[0, 0])


# speedup vs baseline: 1.9335x; 1.1490x over previous
import functools

import jax
import jax.numpy as jnp
from jax.experimental import pallas as pl
from jax.experimental.pallas import tpu as pltpu

N_EMB = 1024
EMB_DIM = 64
ROWS = 2


def _vq_kernel(x_ref, w_ref, q_ref, loss_ref, perp_ref, use_ref,
               counts_ref, dacc_ref, *, n_tokens, grid):
    i = pl.program_id(0)

    @pl.when(i == 0)
    def _init():
        counts_ref[...] = jnp.zeros_like(counts_ref)
        dacc_ref[...] = jnp.zeros_like(dacc_ref)

    w = w_ref[...]
    K = w.shape[0]
    w2 = jnp.sum(w ** 2, axis=1, keepdims=True)
    wneg2 = w * (-2.0)

    for r in range(x_ref.shape[0]):
        xb = x_ref[r]
        x2 = jnp.sum(xb ** 2, axis=0, keepdims=True)
        mm2 = jax.lax.dot_general(wneg2, xb, (((1,), (0,)), ((), ())),
                                  preferred_element_type=jnp.float32)
        d = (w2 + x2) + mm2

        dmin = jnp.min(d, axis=0, keepdims=True)
        iota = jax.lax.broadcasted_iota(jnp.int32, d.shape, 0)
        idxr = jnp.min(jnp.where(d == dmin, iota, K), axis=0, keepdims=True)

        onehot = (iota == idxr).astype(jnp.float32)
        q = jax.lax.dot_general(w, onehot, (((0,), (0,)), ((), ())),
                                preferred_element_type=jnp.float32)
        q_ref[r] = xb + (q - xb)

        counts_ref[...] += jnp.sum(onehot, axis=1, keepdims=True)
        dacc_ref[...] += dmin

    @pl.when(i == grid - 1)
    def _finalize():
        counts = counts_ref[...]
        mse = jnp.sum(dacc_ref[...], axis=1, keepdims=True) \
            * (1.0 / float(n_tokens * EMB_DIM))
        loss_ref[...] = mse + 2.0 * mse
        avg = counts * (1.0 / float(n_tokens))
        ent = jnp.sum(avg * jnp.log(avg + 1e-10), axis=0, keepdims=True)
        perp_ref[...] = jnp.exp(-ent)
        use_ref[...] = jnp.sum((counts >= 1.0).astype(jnp.float32),
                               axis=0, keepdims=True)


@jax.jit
def kernel(x, weight):
    K, D = weight.shape
    B, S, _ = x.shape
    n_tokens = B * S
    grid = B // ROWS
    xt = jnp.swapaxes(x, 1, 2)

    kfn = functools.partial(_vq_kernel, n_tokens=n_tokens, grid=grid)
    out_shapes = (
        jax.ShapeDtypeStruct((B, D, S), jnp.float32),
        jax.ShapeDtypeStruct((1, 1), jnp.float32),
        jax.ShapeDtypeStruct((1, 1), jnp.float32),
        jax.ShapeDtypeStruct((1, 1), jnp.float32),
    )
    scalar_spec = pl.BlockSpec((1, 1), lambda i: (0, 0))
    q_t, loss, perp, use = pl.pallas_call(
        kfn,
        grid=(grid,),
        in_specs=[
            pl.BlockSpec((ROWS, D, S), lambda i: (i, 0, 0)),
            pl.BlockSpec((K, D), lambda i: (0, 0)),
        ],
        out_specs=(
            pl.BlockSpec((ROWS, D, S), lambda i: (i, 0, 0)),
            scalar_spec, scalar_spec, scalar_spec,
        ),
        out_shape=out_shapes,
        scratch_shapes=[
            pltpu.VMEM((K, 1), jnp.float32),
            pltpu.VMEM((1, S), jnp.float32),
        ],
    )(xt, weight)

    return (jnp.swapaxes(q_t, 1, 2), loss[0, 0], perp[0, 0], use[0, 0])


# speedup vs baseline: 2.3249x; 1.2025x over previous
import functools

import jax
import jax.numpy as jnp
from jax.experimental import pallas as pl
from jax.experimental.pallas import tpu as pltpu

N_EMB = 1024
EMB_DIM = 64
ROWS = 2


def _vq_kernel(x_ref, w_ref, q_ref, loss_ref, perp_ref, use_ref,
               counts_ref, dacc_ref, *, n_tokens, grid):
    i = pl.program_id(0)

    @pl.when(i == 0)
    def _init():
        counts_ref[...] = jnp.zeros_like(counts_ref)
        dacc_ref[...] = jnp.zeros_like(dacc_ref)

    w = w_ref[...]
    K = w.shape[0]
    w2 = jnp.sum(w ** 2, axis=1, keepdims=True)
    wneg2 = w * (-2.0)

    for r in range(x_ref.shape[0]):
        xb = x_ref[r]
        x2 = jnp.sum(xb ** 2, axis=0, keepdims=True)
        mm2 = jax.lax.dot_general(wneg2, xb, (((1,), (0,)), ((), ())),
                                  preferred_element_type=jnp.float32)
        d = (w2 + x2) + mm2

        dmin = jnp.min(d, axis=0, keepdims=True)
        iota = jax.lax.broadcasted_iota(jnp.int32, d.shape, 0)
        idxr = jnp.argmin(d, axis=0).reshape(1, -1)

        onehot = (iota == idxr).astype(jnp.float32)
        q = jax.lax.dot_general(w, onehot, (((0,), (0,)), ((), ())),
                                preferred_element_type=jnp.float32)
        q_ref[r] = xb + (q - xb)

        counts_ref[...] += jnp.sum(onehot, axis=1, keepdims=True)
        dacc_ref[...] += dmin

    @pl.when(i == grid - 1)
    def _finalize():
        counts = counts_ref[...]
        mse = jnp.sum(dacc_ref[...], axis=1, keepdims=True) \
            * (1.0 / float(n_tokens * EMB_DIM))
        loss_ref[...] = mse + 2.0 * mse
        avg = counts * (1.0 / float(n_tokens))
        ent = jnp.sum(avg * jnp.log(avg + 1e-10), axis=0, keepdims=True)
        perp_ref[...] = jnp.exp(-ent)
        use_ref[...] = jnp.sum((counts >= 1.0).astype(jnp.float32),
                               axis=0, keepdims=True)


@jax.jit
def kernel(x, weight):
    K, D = weight.shape
    B, S, _ = x.shape
    n_tokens = B * S
    grid = B // ROWS
    xt = jnp.swapaxes(x, 1, 2)

    kfn = functools.partial(_vq_kernel, n_tokens=n_tokens, grid=grid)
    out_shapes = (
        jax.ShapeDtypeStruct((B, D, S), jnp.float32),
        jax.ShapeDtypeStruct((1, 1), jnp.float32),
        jax.ShapeDtypeStruct((1, 1), jnp.float32),
        jax.ShapeDtypeStruct((1, 1), jnp.float32),
    )
    scalar_spec = pl.BlockSpec((1, 1), lambda i: (0, 0))
    q_t, loss, perp, use = pl.pallas_call(
        kfn,
        grid=(grid,),
        in_specs=[
            pl.BlockSpec((ROWS, D, S), lambda i: (i, 0, 0)),
            pl.BlockSpec((K, D), lambda i: (0, 0)),
        ],
        out_specs=(
            pl.BlockSpec((ROWS, D, S), lambda i: (i, 0, 0)),
            scalar_spec, scalar_spec, scalar_spec,
        ),
        out_shape=out_shapes,
        scratch_shapes=[
            pltpu.VMEM((K, 1), jnp.float32),
            pltpu.VMEM((1, S), jnp.float32),
        ],
    )(xt, weight)

    return (jnp.swapaxes(q_t, 1, 2), loss[0, 0], perp[0, 0], use[0, 0])
